# Initial kernel scaffold; baseline (speedup 1.0000x reference)
import jax
import jax.numpy as jnp
from jax.experimental import pallas as pl


def kernel(x, edge_index, kernel_ids, W1, g1, b1, W2, g2, b2):
    raise NotImplementedError("write your pallas kernel here")


# speedup vs baseline: 24.3958x; 24.3958x over previous
import functools

import jax
import jax.numpy as jnp
from jax import lax
from jax.experimental import pallas as pl
from jax.experimental.pallas import tpu as pltpu
from jax.experimental.pallas import tpu_sc as plsc

N = 10000
E = 320000
C = 128
K = 9

NC = 2
NS = 16
NW = NC * NS
EPW = E // NW
G = 100
CH = EPW // G
NP = 10240
RPT = NP // NS

BN = 1000
NB = N // BN


def _mm_body(x_ref, w_ref, y_ref):
    y_ref[0] = jnp.dot(x_ref[...], w_ref[0],
                       preferred_element_type=jnp.float32)


def _transform(h, W):
    return pl.pallas_call(
        _mm_body,
        grid=(K, NB),
        in_specs=[
            pl.BlockSpec((BN, C), lambda k, n: (n, 0)),
            pl.BlockSpec((1, C, C), lambda k, n: (k, 0, 0)),
        ],
        out_specs=pl.BlockSpec((1, BN, C), lambda k, n: (k, n, 0)),
        out_shape=jax.ShapeDtypeStruct((K, N, C), jnp.float32),
    )(h, W)


def _ln(h, g, b):
    mu = jnp.mean(h, axis=-1, keepdims=True)
    var = jnp.mean((h - mu) ** 2, axis=-1, keepdims=True)
    return (h - mu) * lax.rsqrt(var + 1e-6) * g + b


def _mid_body(p_ref, g_ref, b_ref, w_ref, y_ref):
    h = p_ref[0] + p_ref[1]
    h = jax.nn.gelu(_ln(h, g_ref[...], b_ref[...]))
    for k in range(K):
        y_ref[k] = jnp.dot(h, w_ref[k], preferred_element_type=jnp.float32)


def _mid(parts, g, b, W):
    return pl.pallas_call(
        _mid_body,
        grid=(NB,),
        in_specs=[
            pl.BlockSpec((NC, BN, C), lambda n: (0, n, 0)),
            pl.BlockSpec((1, C), lambda n: (0, 0)),
            pl.BlockSpec((1, C), lambda n: (0, 0)),
            pl.BlockSpec((K, C, C), lambda n: (0, 0, 0)),
        ],
        out_specs=pl.BlockSpec((K, BN, C), lambda n: (0, n, 0)),
        out_shape=jax.ShapeDtypeStruct((K, N, C), jnp.float32),
    )(parts, g, b, W)


def _final_body(p_ref, g_ref, b_ref, x_ref, o_ref):
    h = p_ref[0] + p_ref[1]
    h = _ln(h, g_ref[...], b_ref[...]) + x_ref[...]
    o_ref[...] = jax.nn.gelu(h)


def _final(parts, g, b, x):
    return pl.pallas_call(
        _final_body,
        grid=(NB,),
        in_specs=[
            pl.BlockSpec((NC, BN, C), lambda n: (0, n, 0)),
            pl.BlockSpec((1, C), lambda n: (0, 0)),
            pl.BlockSpec((1, C), lambda n: (0, 0)),
            pl.BlockSpec((BN, C), lambda n: (n, 0)),
        ],
        out_specs=pl.BlockSpec((BN, C), lambda n: (n, 0)),
        out_shape=jax.ShapeDtypeStruct((N, C), jnp.float32),
    )(parts, g, b, x)


@functools.cache
def _make_sc_conv():
    mesh = plsc.VectorSubcoreMesh(core_axis_name="c", subcore_axis_name="s")

    @functools.partial(
        pl.kernel,
        out_type=jax.ShapeDtypeStruct((NC, NP, C), jnp.float32),
        mesh=mesh,
        scratch_types=[
            pltpu.VMEM((CH, G), jnp.int32),
            pltpu.VMEM((CH, G), jnp.int32),
            pltpu.VMEM((G, C), jnp.float32),
            pltpu.VMEM_SHARED((NP, C), jnp.float32),
            pltpu.SemaphoreType.DMA,
        ],
    )
    def sc_conv(y_hbm, gidx_hbm, didx_hbm, zeros_hbm, out_hbm,
                gidx_v, didx_v, rows_v, acc_s, sem):
        c = lax.axis_index("c")
        s = lax.axis_index("s")
        wid = s * NC + c
        pltpu.sync_copy(zeros_hbm.at[pl.ds(s * RPT, RPT)],
                        acc_s.at[pl.ds(s * RPT, RPT)])
        pltpu.sync_copy(gidx_hbm.at[wid], gidx_v)
        pltpu.sync_copy(didx_hbm.at[wid], didx_v)
        plsc.subcore_barrier()

        def chunk(j, carry):
            pltpu.async_copy(y_hbm.at[gidx_v.at[j]], rows_v, sem).wait()
            pltpu.sync_copy(rows_v, acc_s.at[didx_v.at[j]], add=True)
            return carry

        lax.fori_loop(0, CH, chunk, 0)
        plsc.subcore_barrier()
        pltpu.sync_copy(acc_s.at[pl.ds(s * RPT, RPT)],
                        out_hbm.at[c, pl.ds(s * RPT, RPT)])

    return sc_conv


def _sc_conv(y, gidx, didx, zeros):
    return _make_sc_conv()(y, gidx, didx, zeros)


def kernel(x, edge_index, kernel_ids, W1, g1, b1, W2, g2, b2):
    src = edge_index[0].astype(jnp.int32)
    dst = edge_index[1].astype(jnp.int32)
    gidx = (kernel_ids.astype(jnp.int32) * N + src).reshape(NW, CH, G)
    didx = dst.reshape(NW, CH, G)
    zeros = jnp.zeros((NP, C), jnp.float32)
    g1r, b1r = g1.reshape(1, C), b1.reshape(1, C)
    g2r, b2r = g2.reshape(1, C), b2.reshape(1, C)

    y1 = _transform(x, W1).reshape(K * N, C)
    p1 = _sc_conv(y1, gidx, didx, zeros)
    y2 = _mid(p1, g1r, b1r, W2).reshape(K * N, C)
    p2 = _sc_conv(y2, gidx, didx, zeros)
    return _final(p2, g2r, b2r, x)
